# baseline (device time: 12472 ns/iter reference)
import jax
import jax.numpy as jnp
from jax import lax
from jax.experimental import pallas as pl
from jax.experimental.pallas import tpu as pltpu


def kernel(partial, gamma):
    _, m, d = partial.shape
    half = m // 2
    chunk = half // 2

    p2d = partial.reshape(m, d)
    g2d = gamma.reshape(1, d)

    def body(p_ref, g_ref, out_ref, comm_ref, send_sem, recv_sem):
        my_x = lax.axis_index("x")
        my_y = lax.axis_index("y")
        my_z = lax.axis_index("z")
        y_partner = (my_x, 1 - my_y, my_z)

        barrier_sem = pltpu.get_barrier_semaphore()
        pl.semaphore_signal(
            barrier_sem, inc=1,
            device_id=y_partner, device_id_type=pl.DeviceIdType.MESH,
        )
        pl.semaphore_wait(barrier_sem, 1)

        my_rows = my_y * half + my_x * chunk
        peer_rows = (1 - my_y) * half + my_x * chunk
        out_base = my_x * chunk

        rdma = pltpu.make_async_remote_copy(
            src_ref=p_ref.at[pl.ds(peer_rows, chunk), :],
            dst_ref=comm_ref,
            send_sem=send_sem,
            recv_sem=recv_sem,
            device_id=y_partner,
            device_id_type=pl.DeviceIdType.MESH,
        )
        rdma.start()
        rdma.wait()

        acc = p_ref[pl.ds(my_rows, chunk), :] + comm_ref[:, :]
        ms = jnp.mean(acc * acc, axis=-1, keepdims=True) + 1e-6
        out_ref[pl.ds(out_base, chunk), :] = acc * lax.rsqrt(ms) * g_ref[:, :]
        out_ref[pl.ds(half - chunk - out_base, chunk), :] = acc

    return pl.pallas_call(
        body,
        out_shape=jax.ShapeDtypeStruct((half, d), jnp.float32),
        in_specs=[
            pl.BlockSpec(memory_space=pltpu.VMEM),
            pl.BlockSpec(memory_space=pltpu.VMEM),
        ],
        out_specs=pl.BlockSpec(memory_space=pltpu.VMEM),
        scratch_shapes=[
            pltpu.VMEM((chunk, d), jnp.float32),
            pltpu.SemaphoreType.DMA,
            pltpu.SemaphoreType.DMA,
        ],
        compiler_params=pltpu.CompilerParams(collective_id=0),
    )(p2d, g2d)
